# baseline (speedup 1.0000x reference)
import functools

import jax
import jax.numpy as jnp
from jax.experimental import pallas as pl
from jax.experimental.pallas import tpu as pltpu

_NEG = float("-inf")


def _chunk_top(x, vi, beam):
    big = jnp.int32(2**30)
    cv, ci = [], []
    for k in range(beam):
        m = jnp.max(x)
        g = jnp.min(jnp.where(x == m, vi, big))
        cv.append(m)
        ci.append(g)
        if k + 1 < beam:
            x = jnp.where(vi == g, _NEG, x)
    return cv, ci


def _body(lg_ref, *refs, n_kv, vocab, beam, cl):
    kv_in = refs[:n_kv]
    probs_ref = refs[n_kv]
    idx_ref = refs[n_kv + 1]
    kv_out = refs[n_kv + 2:n_kv + 2 + n_kv]
    (sems,) = refs[n_kv + 2 + n_kv:]

    def copy(j, b):
        return pltpu.make_async_copy(kv_in[j].at[0], kv_out[j].at[b],
                                     sems.at[j, b])

    for j in range(n_kv):
        for b in range(beam):
            copy(j, b).start(priority=(j * beam + b) % 2)

    cand_v, cand_i = [], []
    cms, css = [], []
    n_chunks = -(-vocab // cl)
    for j in range(n_chunks):
        off = j * cl
        size = min(cl, vocab - off)
        x = lg_ref[:, pl.ds(off, size)]
        vi = jax.lax.broadcasted_iota(jnp.int32, (1, size), 1) + off
        cv, ci = _chunk_top(x, vi, beam)
        cand_v.extend(cv)
        cand_i.extend(ci)
        cms.append(cv[0])
        css.append(jnp.sum(jnp.exp(x - cv[0])))

    big = jnp.int32(2**30)
    nc = len(cand_v)
    lane_c = jax.lax.broadcasted_iota(jnp.int32, (1, nc), 1)
    candv = jnp.zeros((1, nc), jnp.float32)
    candi = jnp.zeros((1, nc), jnp.int32)
    for k in range(nc):
        candv = jnp.where(lane_c == k, cand_v[k], candv)
        candi = jnp.where(lane_c == k, cand_i[k], candi)

    mg = cms[0]
    for c in cms[1:]:
        mg = jnp.maximum(mg, c)
    sg = css[0] * jnp.exp(cms[0] - mg)
    for c, s in zip(cms[1:], css[1:]):
        sg = sg + s * jnp.exp(c - mg)
    lse = jnp.log(sg) + mg

    lane_b = jax.lax.broadcasted_iota(jnp.int32, (1, beam), 1)
    pv = jnp.zeros((1, beam), jnp.float32)
    iv = jnp.zeros((1, beam), jnp.int32)
    for k in range(beam):
        m = jnp.max(candv)
        g = jnp.min(jnp.where(candv == m, candi, big))
        pv = jnp.where(lane_b == k, m - lse, pv)
        iv = jnp.where(lane_b == k, g, iv)
        candv = jnp.where(candi == g, _NEG, candv)
    probs_ref[...] = pv
    idx_ref[...] = iv

    for j in range(n_kv):
        for b in range(beam):
            copy(j, b).wait()


def kernel(kv_0, kv_1, kv_2, kv_3, kv_4, kv_5, kv_6, kv_7, kv_8, kv_9,
           kv_10, kv_11, logits, save_id, beam_size):
    kvs = [kv_0, kv_1, kv_2, kv_3, kv_4, kv_5, kv_6, kv_7, kv_8, kv_9,
           kv_10, kv_11]
    n_kv = len(kvs)
    beam = save_id.shape[0]
    kv_shape = kvs[0].shape
    vocab = logits.shape[-1]

    cl = 83456

    body = functools.partial(_body, n_kv=n_kv, vocab=vocab, beam=beam, cl=cl)
    in_specs = [pl.BlockSpec(memory_space=pltpu.MemorySpace.VMEM)]
    in_specs += [pl.BlockSpec(memory_space=pl.ANY)] * n_kv
    out_specs = [pl.BlockSpec(memory_space=pltpu.MemorySpace.VMEM),
                 pl.BlockSpec(memory_space=pltpu.MemorySpace.VMEM)]
    out_specs += [pl.BlockSpec(memory_space=pl.ANY)] * n_kv
    out_shape = [jax.ShapeDtypeStruct((1, beam), jnp.float32),
                 jax.ShapeDtypeStruct((1, beam), jnp.int32)]
    out_shape += [jax.ShapeDtypeStruct((beam,) + kv_shape[1:],
                                       jnp.float32)] * n_kv

    outs = pl.pallas_call(
        body,
        in_specs=in_specs,
        out_specs=out_specs,
        out_shape=out_shape,
        scratch_shapes=[pltpu.SemaphoreType.DMA((n_kv, beam))],
    )(logits, *kvs)

    probs, idx = outs[0], outs[1]
    kv_outs = list(outs[2:])

    idx_t = idx.reshape(beam, 1)
    save_id_out = jnp.concatenate([save_id, idx_t], axis=-1)
    probs_t = probs.reshape(beam, 1)
    bz = jnp.asarray(beam_size, jnp.int32) - jnp.int32(beam)
    max_idx = idx_t[0] + bz
    return (*kv_outs, idx_t, save_id_out, probs_t, max_idx)


# speedup vs baseline: 24.6749x; 24.6749x over previous
import functools

import jax
import jax.numpy as jnp
from jax.experimental import pallas as pl
from jax.experimental.pallas import tpu as pltpu

_NEG = float("-inf")


def _chunk_top(x, vi, beam):
    big = jnp.int32(2**30)
    cv, ci = [], []
    for k in range(beam):
        m = jnp.max(x)
        g = jnp.min(jnp.where(x == m, vi, big))
        cv.append(m)
        ci.append(g)
        if k + 1 < beam:
            x = jnp.where(vi == g, _NEG, x)
    return cv, ci


def _body(lg_ref, *refs, n_kv, vocab, beam, cl, n_slots):
    kv_in = refs[:n_kv]
    probs_ref = refs[n_kv]
    idx_ref = refs[n_kv + 1]
    kv_out = refs[n_kv + 2:n_kv + 2 + n_kv]
    slots, in_sems, out_sems = refs[n_kv + 2 + n_kv:]

    def in_copy(j):
        return pltpu.make_async_copy(kv_in[j].at[0], slots.at[j % n_slots],
                                     in_sems.at[j % n_slots])

    def out_copy(j, b):
        return pltpu.make_async_copy(slots.at[j % n_slots], kv_out[j].at[b],
                                     out_sems.at[j % n_slots, b])

    cand_v, cand_i = [], []
    cms, css = [], []

    for j in range(min(n_slots - 1, n_kv)):
        in_copy(j).start()

    for j in range(n_kv):
        in_copy(j).wait()
        for b in range(beam):
            out_copy(j, b).start()
        nxt = j + n_slots - 1
        if nxt < n_kv:
            if nxt - n_slots >= 0:
                for b in range(beam):
                    out_copy(nxt - n_slots, b).wait()
            in_copy(nxt).start()

        off = j * cl
        size = min(cl, vocab - off)
        x = lg_ref[:, pl.ds(off, size)]
        vi = jax.lax.broadcasted_iota(jnp.int32, (1, size), 1) + off
        cv, ci = _chunk_top(x, vi, beam)
        cand_v.extend(cv)
        cand_i.extend(ci)
        cms.append(cv[0])
        css.append(jnp.sum(jnp.exp(x - cv[0])))

    for j in range(max(0, n_kv - n_slots), n_kv):
        for b in range(beam):
            out_copy(j, b).wait()

    big = jnp.int32(2**30)
    nc = len(cand_v)
    lane_c = jax.lax.broadcasted_iota(jnp.int32, (1, nc), 1)
    candv = jnp.zeros((1, nc), jnp.float32)
    candi = jnp.zeros((1, nc), jnp.int32)
    for k in range(nc):
        candv = jnp.where(lane_c == k, cand_v[k], candv)
        candi = jnp.where(lane_c == k, cand_i[k], candi)

    mg = cms[0]
    for c in cms[1:]:
        mg = jnp.maximum(mg, c)
    sg = css[0] * jnp.exp(cms[0] - mg)
    for c, s in zip(cms[1:], css[1:]):
        sg = sg + s * jnp.exp(c - mg)
    lse = jnp.log(sg) + mg

    lane_b = jax.lax.broadcasted_iota(jnp.int32, (1, beam), 1)
    pv = jnp.zeros((1, beam), jnp.float32)
    iv = jnp.zeros((1, beam), jnp.int32)
    for k in range(beam):
        m = jnp.max(candv)
        g = jnp.min(jnp.where(candv == m, candi, big))
        pv = jnp.where(lane_b == k, m - lse, pv)
        iv = jnp.where(lane_b == k, g, iv)
        candv = jnp.where(candi == g, _NEG, candv)
    probs_ref[...] = pv
    idx_ref[...] = iv


def kernel(kv_0, kv_1, kv_2, kv_3, kv_4, kv_5, kv_6, kv_7, kv_8, kv_9,
           kv_10, kv_11, logits, save_id, beam_size):
    kvs = [kv_0, kv_1, kv_2, kv_3, kv_4, kv_5, kv_6, kv_7, kv_8, kv_9,
           kv_10, kv_11]
    n_kv = len(kvs)
    beam = save_id.shape[0]
    kv_shape = kvs[0].shape
    vocab = logits.shape[-1]

    cl = 83456
    assert (n_kv - 1) * cl < vocab <= n_kv * cl
    n_slots = 2

    body = functools.partial(_body, n_kv=n_kv, vocab=vocab, beam=beam,
                             cl=cl, n_slots=n_slots)
    in_specs = [pl.BlockSpec(memory_space=pltpu.MemorySpace.VMEM)]
    in_specs += [pl.BlockSpec(memory_space=pl.ANY)] * n_kv
    out_specs = [pl.BlockSpec(memory_space=pltpu.MemorySpace.VMEM),
                 pl.BlockSpec(memory_space=pltpu.MemorySpace.VMEM)]
    out_specs += [pl.BlockSpec(memory_space=pl.ANY)] * n_kv
    out_shape = [jax.ShapeDtypeStruct((1, beam), jnp.float32),
                 jax.ShapeDtypeStruct((1, beam), jnp.int32)]
    out_shape += [jax.ShapeDtypeStruct((beam,) + kv_shape[1:],
                                       jnp.float32)] * n_kv

    outs = pl.pallas_call(
        body,
        in_specs=in_specs,
        out_specs=out_specs,
        out_shape=out_shape,
        scratch_shapes=[pltpu.VMEM((n_slots,) + kv_shape[1:], jnp.float32),
                        pltpu.SemaphoreType.DMA((n_slots,)),
                        pltpu.SemaphoreType.DMA((n_slots, beam))],
    )(logits, *kvs)

    probs, idx = outs[0], outs[1]
    kv_outs = list(outs[2:])

    idx_t = idx.reshape(beam, 1)
    save_id_out = jnp.concatenate([save_id, idx_t], axis=-1)
    probs_t = probs.reshape(beam, 1)
    bz = jnp.asarray(beam_size, jnp.int32) - jnp.int32(beam)
    max_idx = idx_t[0] + bz
    return (*kv_outs, idx_t, save_id_out, probs_t, max_idx)
